# baseline (speedup 1.0000x reference)
import functools

import jax
import jax.numpy as jnp
from jax import lax
from jax.experimental import pallas as pl
from jax.experimental.pallas import tpu as pltpu
from jax.experimental.pallas import tpu_sc as plsc

_N = 10000
_E = 320000
_DIN = 128
_DH = 32
_DOUT = 2
_G = 64
_DFC = 128

_NC = 2
_NS = 16
_NW = _NC * _NS

_C = 125
_K = 8
_EROWS = _E // _C
_RPT = _EROWS // _NW
_RPTF = _EROWS // _NS
_NOUT = _RPT // _K
_NOUTF = _RPTF // _K

_NP = 10240
_RNODE = _NP // _NS
_NPK = _NP // 4

_mesh = plsc.VectorSubcoreMesh(core_axis_name="c", subcore_axis_name="s")


@functools.partial(
    pl.kernel,
    out_type=jax.ShapeDtypeStruct((_NP, _DH), jnp.float32),
    mesh=_mesh,
    scratch_types=[
        [pltpu.VMEM((_K, _C), jnp.int32)] * 3,
        pltpu.VMEM((_RNODE,), jnp.float32),
        pltpu.VMEM((_RNODE,), jnp.float32),
        pltpu.VMEM((_RNODE, _DH), jnp.float32),
        pltpu.VMEM_SHARED((_NP,), jnp.float32),
        pltpu.SemaphoreType.DMA,
        pltpu.SemaphoreType.DMA,
    ],
    compiler_params=pltpu.CompilerParams(use_tc_tiling_on_sc=False,
                                         needs_layout_passes=False),
)
def _sc_degree(dst2d, dinvb, didx, ones_v, dbuf, dbv, acc, sem_i, sem_s):
    s = lax.axis_index("s")

    def zfill(j, carry):
        ones_v[pl.ds(j * 16, 16)] = jnp.zeros((16,), jnp.float32)
        return carry

    lax.fori_loop(0, _RNODE // 16, zfill, 0)
    pltpu.sync_copy(ones_v, acc.at[pl.ds(s * _RNODE, _RNODE)])
    for j in range(128 // 16):
        ones_v[pl.ds(j * 16, 16)] = jnp.ones((16,), jnp.float32)
    plsc.subcore_barrier()

    base = s * _RPTF
    ones_s = ones_v.at[pl.ds(0, _C)]

    idx_d = [None] * (_NOUTF + 1)
    sc_d = [None] * _NOUTF
    idx_d[0] = pltpu.async_copy(dst2d.at[pl.ds(base, _K), :], didx[0], sem_i)
    for i in range(_NOUTF):
        ib = didx[i % 3]
        if i >= 2:
            for d in sc_d[i - 2]:
                d.wait()
        idx_d[i].wait()
        if i + 1 < _NOUTF:
            idx_d[i + 1] = pltpu.async_copy(
                dst2d.at[pl.ds(base + (i + 1) * _K, _K), :],
                didx[(i + 1) % 3], sem_i)
        sc_d[i] = [pltpu.async_copy(ones_s, acc.at[ib.at[j]], sem_s,
                                    add=True)
                   for j in range(_K)]
    for i in (_NOUTF - 2, _NOUTF - 1):
        for d in sc_d[i]:
            d.wait()

    plsc.subcore_barrier()

    pltpu.sync_copy(acc.at[pl.ds(s * _RNODE, _RNODE)], dbuf)

    def newton(k, carry):
        v = dbuf[pl.ds(k * 16, 16)] + 1.0
        i0 = plsc.bitcast(v, jnp.int32)
        i0 = 0x5F3759DF - lax.shift_right_logical(i0, 1)
        y = plsc.bitcast(i0, jnp.float32)
        y = y * (1.5 - 0.5 * v * y * y)
        y = y * (1.5 - 0.5 * v * y * y)
        y = y * (1.5 - 0.5 * v * y * y)
        dbuf[pl.ds(k * 16, 16)] = y
        return carry

    lax.fori_loop(0, _RNODE // 16, newton, 0)

    def expand(k, carry):
        dv = dbuf[pl.ds(k * 16, 16)]
        for l in range(16):
            row = jnp.broadcast_to(dv[l], (16,))
            dbv[k * 16 + l, pl.ds(0, 16)] = row
            dbv[k * 16 + l, pl.ds(16, 16)] = row
        return carry

    lax.fori_loop(0, _RNODE // 16, expand, 0)
    pltpu.sync_copy(dbv, dinvb.at[pl.ds(s * _RNODE, _RNODE), :])


@functools.partial(
    pl.kernel,
    out_type=jax.ShapeDtypeStruct((_NC * _NP, _DH), jnp.float32),
    mesh=_mesh,
    scratch_types=[
        [pltpu.VMEM((2, _K, _C), jnp.int32)] * 3,
        [pltpu.VMEM((_K, _C, _DH), jnp.float32)] * 2,
        pltpu.VMEM((80, _DH), jnp.float32),
        pltpu.VMEM_SHARED((_NP, _DH), jnp.float32),
        pltpu.SemaphoreType.DMA,
        pltpu.SemaphoreType.DMA,
        pltpu.SemaphoreType.DMA,
    ],
    compiler_params=pltpu.CompilerParams(use_tc_tiling_on_sc=False),
)
def _sc_message(src2d, dst2d, table, out, ibuf, rows, zbuf, acc,
                sem_i, sem_g, sem_s):
    c = lax.axis_index("c")
    s = lax.axis_index("s")
    wid = s * _NC + c

    def zfill(k, carry):
        zbuf[k, pl.ds(0, 16)] = jnp.zeros((16,), jnp.float32)
        zbuf[k, pl.ds(16, 16)] = jnp.zeros((16,), jnp.float32)
        return carry

    lax.fori_loop(0, 80, zfill, 0)
    for t in range(_RNODE // 80):
        pltpu.sync_copy(zbuf, acc.at[pl.ds(s * _RNODE + t * 80, 80), :])
    plsc.subcore_barrier()

    base = wid * _RPT

    def idx_start(i, buf):
        return [pltpu.async_copy(src2d.at[pl.ds(base + i * _K, _K), :],
                                 buf.at[0], sem_i),
                pltpu.async_copy(dst2d.at[pl.ds(base + i * _K, _K), :],
                                 buf.at[1], sem_i)]

    idx_d = [None] * (_NOUT + 1)
    sc_d = [None] * _NOUT
    idx_d[0] = idx_start(0, ibuf[0])
    for i in range(_NOUT):
        rb = rows[i % 2]
        ib = ibuf[i % 3]
        if i >= 2:
            for d in sc_d[i - 2]:
                d.wait()
        for d in idx_d[i]:
            d.wait()
        if i + 1 < _NOUT:
            idx_d[i + 1] = idx_start(i + 1, ibuf[(i + 1) % 3])
        gs = [pltpu.async_copy(table.at[ib.at[0, j]], rb.at[j], sem_g)
              for j in range(_K)]
        for g in gs:
            g.wait()
        sc_d[i] = [pltpu.async_copy(rb.at[j], acc.at[ib.at[1, j]], sem_s,
                                    add=True)
                   for j in range(_K)]
    for i in (_NOUT - 2, _NOUT - 1):
        for d in sc_d[i]:
            d.wait()

    plsc.subcore_barrier()
    pltpu.sync_copy(acc.at[pl.ds(s * _RNODE, _RNODE), :],
                    out.at[pl.ds(c * _NP + s * _RNODE, _RNODE), :])


_BPK = 512
_GRIDK = _NPK // _BPK


def _tc1_body(dinvb_ref, xpk_ref, w1b_ref, out_ref):
    out_ref[...] = dinvb_ref[...] * jnp.dot(xpk_ref[...], w1b_ref[...])


def _tc1(dinvb_p, xpk, W1blk):
    return pl.pallas_call(
        _tc1_body,
        grid=(_GRIDK,),
        in_specs=[
            pl.BlockSpec((_BPK, 128), lambda i: (i, 0)),
            pl.BlockSpec((_BPK, 4 * _DIN), lambda i: (i, 0)),
            pl.BlockSpec((4 * _DIN, 128), lambda i: (0, 0)),
        ],
        out_specs=pl.BlockSpec((_BPK, 128), lambda i: (i, 0)),
        out_shape=jax.ShapeDtypeStruct((_NPK, 128), jnp.float32),
    )(dinvb_p, xpk, W1blk)


def _tc2_body(a0_ref, a1_ref, xwp_ref, dinvb_ref, b1p_ref, w2b_ref, out_ref):
    d = dinvb_ref[...]
    h = d * (a0_ref[...] + a1_ref[...] + xwp_ref[...]) + b1p_ref[...]
    h = jnp.maximum(h, 0.0)
    out_ref[...] = d * jnp.dot(h, w2b_ref[...])


def _tc2(a0p, a1p, xwp_p, dinvb_p, b1p, W2blk):
    bs = pl.BlockSpec((_BPK, 128), lambda i: (i, 0))
    return pl.pallas_call(
        _tc2_body,
        grid=(_GRIDK,),
        in_specs=[bs, bs, bs, bs,
                  pl.BlockSpec((1, 128), lambda i: (0, 0)),
                  pl.BlockSpec((128, 128), lambda i: (0, 0))],
        out_specs=bs,
        out_shape=jax.ShapeDtypeStruct((_NPK, 128), jnp.float32),
    )(a0p, a1p, xwp_p, dinvb_p, b1p, W2blk)


def _tc3_body(a0_ref, a1_ref, xwp_ref, dinvb_ref, b2p_ref, bi4_ref,
              wf1_ref, bf1_ref, wf2_ref, bf2_ref, out_ref):
    h2 = (dinvb_ref[...] * (a0_ref[...] + a1_ref[...] + xwp_ref[...])
          + b2p_ref[...])
    h2 = jnp.maximum(h2, 0.0)
    gidx = lax.broadcasted_iota(jnp.int32, (_NPK, _G), 1)
    ones = jnp.ones((_NPK, 1), jnp.float32)
    dn = (((0,), (0,)), ((), ()))
    hp = lax.Precision.HIGHEST
    ssum = jnp.zeros((_G, _DH), jnp.float32)
    cnt = jnp.zeros((_G, 1), jnp.float32)
    for q in range(4):
        oh = (bi4_ref[:, q:q + 1] == gidx).astype(jnp.float32)
        ssum = ssum + lax.dot_general(oh, h2[:, 32 * q:32 * q + 32], dn,
                                      precision=hp)
        cnt = cnt + lax.dot_general(oh, ones, dn, precision=hp)
    gemb = ssum / jnp.maximum(cnt, 1.0)
    z = jnp.dot(gemb, wf1_ref[...]) + bf1_ref[...]
    z = jnp.maximum(z, 0.0)
    out_ref[...] = jnp.dot(z, wf2_ref[...]) + bf2_ref[...]


def _tc3(a0p, a1p, xwp_p, dinvb_p, b2p, bi4, Wf1, bf1, Wf2, bf2):
    return pl.pallas_call(
        _tc3_body,
        out_shape=jax.ShapeDtypeStruct((_G, _DOUT), jnp.float32),
    )(a0p, a1p, xwp_p, dinvb_p, b2p, bi4, Wf1, bf1, Wf2, bf2)


@jax.jit
def kernel(x, edge_index, batch_index, W1, b1, W2, b2, Wf1, bf1, Wf2, bf2):
    f32 = jnp.float32
    src2d = edge_index[0].reshape(_EROWS, _C)
    dst2d = edge_index[1].reshape(_EROWS, _C)

    pad = _NP - _N
    xpk = jnp.concatenate([x, jnp.zeros((pad, _DIN), f32)],
                          axis=0).reshape(_NPK, 4 * _DIN)
    bi4 = jnp.concatenate(
        [batch_index, jnp.full((pad,), _G, batch_index.dtype)]).reshape(
            _NPK, 4)
    eye4 = jnp.eye(4, dtype=f32)
    W1blk = jnp.kron(eye4, W1)
    W2blk = jnp.kron(eye4, W2)
    b1p = jnp.tile(b1, 4).reshape(1, 128)
    b2p = jnp.tile(b2, 4).reshape(1, 128)

    dinvb = _sc_degree(dst2d)
    dinvb_p = dinvb.reshape(_NPK, 128)

    xwp1_p = _tc1(dinvb_p, xpk, W1blk)
    acc1 = _sc_message(src2d, dst2d, xwp1_p.reshape(_NP, _DH))
    xwp2_p = _tc2(acc1[:_NP].reshape(_NPK, 128),
                  acc1[_NP:].reshape(_NPK, 128),
                  xwp1_p, dinvb_p, b1p, W2blk)
    acc2 = _sc_message(src2d, dst2d, xwp2_p.reshape(_NP, _DH))
    out = _tc3(acc2[:_NP].reshape(_NPK, 128),
               acc2[_NP:].reshape(_NPK, 128),
               xwp2_p, dinvb_p, b2p, bi4,
               Wf1, bf1.reshape(1, _DFC), Wf2, bf2.reshape(1, _DOUT))
    return out


# speedup vs baseline: 1.3084x; 1.3084x over previous
import functools

import jax
import jax.numpy as jnp
from jax import lax
from jax.experimental import pallas as pl
from jax.experimental.pallas import tpu as pltpu
from jax.experimental.pallas import tpu_sc as plsc

_N = 10000
_E = 320000
_DIN = 128
_DH = 32
_DOUT = 2
_G = 64
_DFC = 128

_NC = 2
_NS = 16
_NW = _NC * _NS

_C = 125
_K = 8
_EROWS = _E // _C
_RPT = _EROWS // _NW
_RPTF = _EROWS // _NS
_NOUT = _RPT // _K
_NOUTF = _RPTF // _K

_NP = 10240
_RNODE = _NP // _NS
_NPK = _NP // 4

_mesh = plsc.VectorSubcoreMesh(core_axis_name="c", subcore_axis_name="s")


@functools.partial(
    pl.kernel,
    out_type=jax.ShapeDtypeStruct((_NPK, 128), jnp.float32),
    mesh=_mesh,
    scratch_types=[
        [pltpu.VMEM((_K, _C), jnp.int32)] * 3,
        pltpu.VMEM((_RNODE,), jnp.float32),
        pltpu.VMEM((_RNODE,), jnp.float32),
        pltpu.VMEM((_RNODE // 4, 128), jnp.float32),
        pltpu.VMEM_SHARED((_NP,), jnp.float32),
        pltpu.SemaphoreType.DMA,
        pltpu.SemaphoreType.DMA,
    ],
    compiler_params=pltpu.CompilerParams(use_tc_tiling_on_sc=False,
                                         needs_layout_passes=False),
)
def _sc_degree(dst2d, dinvb, didx, ones_v, dbuf, dbv, acc, sem_i, sem_s):
    s = lax.axis_index("s")

    def zfill(j, carry):
        ones_v[pl.ds(j * 16, 16)] = jnp.zeros((16,), jnp.float32)
        return carry

    lax.fori_loop(0, _RNODE // 16, zfill, 0)
    pltpu.sync_copy(ones_v, acc.at[pl.ds(s * _RNODE, _RNODE)])
    for j in range(128 // 16):
        ones_v[pl.ds(j * 16, 16)] = jnp.ones((16,), jnp.float32)
    plsc.subcore_barrier()

    base = s * _RPTF
    ones_s = ones_v.at[pl.ds(0, _C)]

    idx_d = [None] * (_NOUTF + 1)
    sc_d = [None] * _NOUTF
    idx_d[0] = pltpu.async_copy(dst2d.at[pl.ds(base, _K), :], didx[0], sem_i)
    for i in range(_NOUTF):
        ib = didx[i % 3]
        if i >= 2:
            for d in sc_d[i - 2]:
                d.wait()
        idx_d[i].wait()
        if i + 1 < _NOUTF:
            idx_d[i + 1] = pltpu.async_copy(
                dst2d.at[pl.ds(base + (i + 1) * _K, _K), :],
                didx[(i + 1) % 3], sem_i)
        sc_d[i] = [pltpu.async_copy(ones_s, acc.at[ib.at[j]], sem_s,
                                    add=True)
                   for j in range(_K)]
    for i in (_NOUTF - 2, _NOUTF - 1):
        for d in sc_d[i]:
            d.wait()

    plsc.subcore_barrier()

    pltpu.sync_copy(acc.at[pl.ds(s * _RNODE, _RNODE)], dbuf)

    def newton(k, carry):
        v = dbuf[pl.ds(k * 16, 16)] + 1.0
        i0 = plsc.bitcast(v, jnp.int32)
        i0 = 0x5F3759DF - lax.shift_right_logical(i0, 1)
        y = plsc.bitcast(i0, jnp.float32)
        y = y * (1.5 - 0.5 * v * y * y)
        y = y * (1.5 - 0.5 * v * y * y)
        y = y * (1.5 - 0.5 * v * y * y)
        dbuf[pl.ds(k * 16, 16)] = y
        return carry

    lax.fori_loop(0, _RNODE // 16, newton, 0)

    def expand(k, carry):
        dv = dbuf[pl.ds(k * 16, 16)]
        for l in range(16):
            row = jnp.broadcast_to(dv[l], (16,))
            dbv[k * 4 + l // 4, pl.ds(32 * (l % 4), 16)] = row
            dbv[k * 4 + l // 4, pl.ds(32 * (l % 4) + 16, 16)] = row
        return carry

    lax.fori_loop(0, _RNODE // 16, expand, 0)
    pltpu.sync_copy(
        dbv, dinvb.at[pl.ds(s * (_RNODE // 4), _RNODE // 4), :])


@functools.partial(
    pl.kernel,
    out_type=jax.ShapeDtypeStruct((_NC * _NP, _DH), jnp.float32),
    mesh=_mesh,
    scratch_types=[
        [pltpu.VMEM((2, _K, _C), jnp.int32)] * 3,
        [pltpu.VMEM((_K, _C, _DH), jnp.float32)] * 2,
        pltpu.VMEM((80, _DH), jnp.float32),
        pltpu.VMEM_SHARED((_NP, _DH), jnp.float32),
        pltpu.SemaphoreType.DMA,
        pltpu.SemaphoreType.DMA,
        pltpu.SemaphoreType.DMA,
    ],
    compiler_params=pltpu.CompilerParams(use_tc_tiling_on_sc=False),
)
def _sc_message(src2d, dst2d, table, out, ibuf, rows, zbuf, acc,
                sem_i, sem_g, sem_s):
    c = lax.axis_index("c")
    s = lax.axis_index("s")
    wid = s * _NC + c

    def zfill(k, carry):
        zbuf[k, pl.ds(0, 16)] = jnp.zeros((16,), jnp.float32)
        zbuf[k, pl.ds(16, 16)] = jnp.zeros((16,), jnp.float32)
        return carry

    lax.fori_loop(0, 80, zfill, 0)
    for t in range(_RNODE // 80):
        pltpu.sync_copy(zbuf, acc.at[pl.ds(s * _RNODE + t * 80, 80), :])
    plsc.subcore_barrier()

    base = wid * _RPT

    def idx_start(i, buf):
        return [pltpu.async_copy(src2d.at[pl.ds(base + i * _K, _K), :],
                                 buf.at[0], sem_i),
                pltpu.async_copy(dst2d.at[pl.ds(base + i * _K, _K), :],
                                 buf.at[1], sem_i)]

    idx_d = [None] * (_NOUT + 1)
    sc_d = [None] * _NOUT
    idx_d[0] = idx_start(0, ibuf[0])
    for i in range(_NOUT):
        rb = rows[i % 2]
        ib = ibuf[i % 3]
        if i >= 2:
            for d in sc_d[i - 2]:
                d.wait()
        for d in idx_d[i]:
            d.wait()
        if i + 1 < _NOUT:
            idx_d[i + 1] = idx_start(i + 1, ibuf[(i + 1) % 3])
        gs = [pltpu.async_copy(table.at[ib.at[0, j]], rb.at[j], sem_g)
              for j in range(_K)]
        for g in gs:
            g.wait()
        sc_d[i] = [pltpu.async_copy(rb.at[j], acc.at[ib.at[1, j]], sem_s,
                                    add=True)
                   for j in range(_K)]
    for i in (_NOUT - 2, _NOUT - 1):
        for d in sc_d[i]:
            d.wait()

    plsc.subcore_barrier()
    pltpu.sync_copy(acc.at[pl.ds(s * _RNODE, _RNODE), :],
                    out.at[pl.ds(c * _NP + s * _RNODE, _RNODE), :])


_BPK = 512
_GRIDK = _NPK // _BPK


def _tc1_body(dinvb_ref, xpk_ref, w1b_ref, out_ref):
    out_ref[...] = dinvb_ref[...] * jnp.dot(xpk_ref[...], w1b_ref[...])


def _tc1(dinvb_p, xpk, W1blk):
    return pl.pallas_call(
        _tc1_body,
        grid=(_GRIDK,),
        in_specs=[
            pl.BlockSpec((_BPK, 128), lambda i: (i, 0)),
            pl.BlockSpec((_BPK, 4 * _DIN), lambda i: (i, 0)),
            pl.BlockSpec((4 * _DIN, 128), lambda i: (0, 0)),
        ],
        out_specs=pl.BlockSpec((_BPK, 128), lambda i: (i, 0)),
        out_shape=jax.ShapeDtypeStruct((_NPK, 128), jnp.float32),
    )(dinvb_p, xpk, W1blk)


def _tc2_body(a0_ref, a1_ref, xwp_ref, dinvb_ref, b1p_ref, w2b_ref, out_ref):
    d = dinvb_ref[...]
    h = d * (a0_ref[...] + a1_ref[...] + xwp_ref[...]) + b1p_ref[...]
    h = jnp.maximum(h, 0.0)
    out_ref[...] = d * jnp.dot(h, w2b_ref[...])


def _tc2(accs, xwp_p, dinvb_p, b1p, W2blk):
    bs = pl.BlockSpec((_BPK, 128), lambda i: (i, 0))
    return pl.pallas_call(
        _tc2_body,
        grid=(_GRIDK,),
        in_specs=[bs,
                  pl.BlockSpec((_BPK, 128), lambda i: (i + _GRIDK, 0)),
                  bs, bs,
                  pl.BlockSpec((1, 128), lambda i: (0, 0)),
                  pl.BlockSpec((128, 128), lambda i: (0, 0))],
        out_specs=bs,
        out_shape=jax.ShapeDtypeStruct((_NPK, 128), jnp.float32),
    )(accs, accs, xwp_p, dinvb_p, b1p, W2blk)


def _tc3_body(a0_ref, a1_ref, xwp_ref, dinvb_ref, b2p_ref, bi4_ref,
              wf1_ref, bf1_ref, wf2_ref, bf2_ref, out_ref):
    h2 = (dinvb_ref[...] * (a0_ref[...] + a1_ref[...] + xwp_ref[...])
          + b2p_ref[...])
    h2 = jnp.maximum(h2, 0.0)
    gidx = lax.broadcasted_iota(jnp.int32, (_NPK, _G), 1)
    ones = jnp.ones((_NPK, 1), jnp.float32)
    dn = (((0,), (0,)), ((), ()))
    hp = lax.Precision.HIGHEST
    ssum = jnp.zeros((_G, _DH), jnp.float32)
    cnt = jnp.zeros((_G, 1), jnp.float32)
    for q in range(4):
        oh = (bi4_ref[:, q:q + 1] == gidx).astype(jnp.float32)
        ssum = ssum + lax.dot_general(oh, h2[:, 32 * q:32 * q + 32], dn,
                                      precision=hp)
        cnt = cnt + lax.dot_general(oh, ones, dn, precision=hp)
    gemb = ssum / jnp.maximum(cnt, 1.0)
    z = jnp.dot(gemb, wf1_ref[...]) + bf1_ref[...]
    z = jnp.maximum(z, 0.0)
    out_ref[...] = jnp.dot(z, wf2_ref[...]) + bf2_ref[...]


def _tc3(accs, xwp_p, dinvb_p, b2p, bi4, Wf1, bf1, Wf2, bf2):
    half = pl.BlockSpec((_NPK, 128), lambda i: (0, 0))
    half1 = pl.BlockSpec((_NPK, 128), lambda i: (1, 0))
    return pl.pallas_call(
        _tc3_body,
        grid=(1,),
        in_specs=[half, half1] + [pl.BlockSpec(a.shape, lambda i: (0, 0))
                                  for a in (xwp_p, dinvb_p, b2p, bi4,
                                            Wf1, bf1, Wf2, bf2)],
        out_specs=pl.BlockSpec((_G, _DOUT), lambda i: (0, 0)),
        out_shape=jax.ShapeDtypeStruct((_G, _DOUT), jnp.float32),
    )(accs, accs, xwp_p, dinvb_p, b2p, bi4, Wf1, bf1, Wf2, bf2)


@jax.jit
def kernel(x, edge_index, batch_index, W1, b1, W2, b2, Wf1, bf1, Wf2, bf2):
    f32 = jnp.float32
    src2d = edge_index[0].reshape(_EROWS, _C)
    dst2d = edge_index[1].reshape(_EROWS, _C)

    pad = _NP - _N
    xpk = jnp.concatenate([x, jnp.zeros((pad, _DIN), f32)],
                          axis=0).reshape(_NPK, 4 * _DIN)
    bi4 = jnp.concatenate(
        [batch_index, jnp.full((pad,), _G, batch_index.dtype)]).reshape(
            _NPK, 4)
    eye4 = jnp.eye(4, dtype=f32)
    W1blk = jnp.kron(eye4, W1)
    W2blk = jnp.kron(eye4, W2)
    b1p = jnp.tile(b1, 4).reshape(1, 128)
    b2p = jnp.tile(b2, 4).reshape(1, 128)

    dinvb_p = _sc_degree(dst2d)

    xwp1_p = _tc1(dinvb_p, xpk, W1blk)
    acc1 = _sc_message(src2d, dst2d, xwp1_p.reshape(_NP, _DH))
    xwp2_p = _tc2(acc1.reshape(_NC * _NPK, 128), xwp1_p, dinvb_p, b1p, W2blk)
    acc2 = _sc_message(src2d, dst2d, xwp2_p.reshape(_NP, _DH))
    out = _tc3(acc2.reshape(_NC * _NPK, 128), xwp2_p, dinvb_p, b2p, bi4,
               Wf1, bf1.reshape(1, _DFC), Wf2, bf2.reshape(1, _DOUT))
    return out


# speedup vs baseline: 1.3144x; 1.0046x over previous
import functools

import jax
import jax.numpy as jnp
from jax import lax
from jax.experimental import pallas as pl
from jax.experimental.pallas import tpu as pltpu
from jax.experimental.pallas import tpu_sc as plsc

_N = 10000
_E = 320000
_DIN = 128
_DH = 32
_DOUT = 2
_G = 64
_DFC = 128

_NC = 2
_NS = 16
_NW = _NC * _NS

_C = 125
_K = 8
_EROWS = _E // _C
_RPT = _EROWS // _NW
_RPTF = _EROWS // _NS
_NOUT = _RPT // _K
_NOUTF = _RPTF // _K

_NP = 10240
_RNODE = _NP // _NS
_NPK = _NP // 4

_mesh = plsc.VectorSubcoreMesh(core_axis_name="c", subcore_axis_name="s")


@functools.partial(
    pl.kernel,
    out_type=jax.ShapeDtypeStruct((_NPK, 128), jnp.float32),
    mesh=_mesh,
    scratch_types=[
        [pltpu.VMEM((_K, _C), jnp.int32)] * 3,
        pltpu.VMEM((_RNODE,), jnp.float32),
        pltpu.VMEM((_RNODE,), jnp.float32),
        pltpu.VMEM((_RNODE // 4, 128), jnp.float32),
        pltpu.VMEM_SHARED((_NP,), jnp.float32),
        pltpu.SemaphoreType.DMA,
        pltpu.SemaphoreType.DMA,
    ],
    compiler_params=pltpu.CompilerParams(use_tc_tiling_on_sc=False,
                                         needs_layout_passes=False),
)
def _sc_degree(dst2d, dinvb, didx, ones_v, dbuf, dbv, acc, sem_i, sem_s):
    s = lax.axis_index("s")

    def zfill(j, carry):
        ones_v[pl.ds(j * 16, 16)] = jnp.zeros((16,), jnp.float32)
        return carry

    lax.fori_loop(0, _RNODE // 16, zfill, 0)
    pltpu.sync_copy(ones_v, acc.at[pl.ds(s * _RNODE, _RNODE)])
    for j in range(128 // 16):
        ones_v[pl.ds(j * 16, 16)] = jnp.ones((16,), jnp.float32)
    plsc.subcore_barrier()

    base = s * _RPTF
    ones_s = ones_v.at[pl.ds(0, _C)]

    idx_d = [None] * (_NOUTF + 1)
    sc_d = [None] * _NOUTF
    idx_d[0] = pltpu.async_copy(dst2d.at[pl.ds(base, _K), :], didx[0], sem_i)
    for i in range(_NOUTF):
        ib = didx[i % 3]
        if i >= 2:
            for d in sc_d[i - 2]:
                d.wait()
        idx_d[i].wait()
        if i + 1 < _NOUTF:
            idx_d[i + 1] = pltpu.async_copy(
                dst2d.at[pl.ds(base + (i + 1) * _K, _K), :],
                didx[(i + 1) % 3], sem_i)
        sc_d[i] = [pltpu.async_copy(ones_s, acc.at[ib.at[j]], sem_s,
                                    add=True)
                   for j in range(_K)]
    for i in (_NOUTF - 2, _NOUTF - 1):
        for d in sc_d[i]:
            d.wait()

    plsc.subcore_barrier()

    pltpu.sync_copy(acc.at[pl.ds(s * _RNODE, _RNODE)], dbuf)

    def newton(k, carry):
        v = dbuf[pl.ds(k * 16, 16)] + 1.0
        i0 = plsc.bitcast(v, jnp.int32)
        i0 = 0x5F3759DF - lax.shift_right_logical(i0, 1)
        y = plsc.bitcast(i0, jnp.float32)
        y = y * (1.5 - 0.5 * v * y * y)
        y = y * (1.5 - 0.5 * v * y * y)
        y = y * (1.5 - 0.5 * v * y * y)
        dbuf[pl.ds(k * 16, 16)] = y
        return carry

    lax.fori_loop(0, _RNODE // 16, newton, 0)

    def expand(k, carry):
        dv = dbuf[pl.ds(k * 16, 16)]
        for l in range(16):
            row = jnp.broadcast_to(dv[l], (16,))
            dbv[k * 4 + l // 4, pl.ds(32 * (l % 4), 16)] = row
            dbv[k * 4 + l // 4, pl.ds(32 * (l % 4) + 16, 16)] = row
        return carry

    lax.fori_loop(0, _RNODE // 16, expand, 0)
    pltpu.sync_copy(
        dbv, dinvb.at[pl.ds(s * (_RNODE // 4), _RNODE // 4), :])


@functools.partial(
    pl.kernel,
    out_type=jax.ShapeDtypeStruct((_NC * _NP, _DH), jnp.float32),
    mesh=_mesh,
    scratch_types=[
        [pltpu.VMEM((2, _K, _C), jnp.int32)] * 3,
        [pltpu.VMEM((_K, _C, _DH), jnp.float32)] * 2,
        pltpu.VMEM((80, _DH), jnp.float32),
        pltpu.VMEM_SHARED((_NP, _DH), jnp.float32),
        pltpu.SemaphoreType.DMA,
        pltpu.SemaphoreType.DMA,
        pltpu.SemaphoreType.DMA,
    ],
    compiler_params=pltpu.CompilerParams(use_tc_tiling_on_sc=False),
)
def _sc_message(src2d, dst2d, table, out, ibuf, rows, zbuf, acc,
                sem_i, sem_g, sem_s):
    c = lax.axis_index("c")
    s = lax.axis_index("s")
    wid = s * _NC + c

    def zfill(k, carry):
        zbuf[k, pl.ds(0, 16)] = jnp.zeros((16,), jnp.float32)
        zbuf[k, pl.ds(16, 16)] = jnp.zeros((16,), jnp.float32)
        return carry

    lax.fori_loop(0, 80, zfill, 0)
    for t in range(_RNODE // 80):
        pltpu.sync_copy(zbuf, acc.at[pl.ds(s * _RNODE + t * 80, 80), :])
    plsc.subcore_barrier()

    base = wid * _RPT

    def idx_start(i, buf):
        return [pltpu.async_copy(src2d.at[pl.ds(base + i * _K, _K), :],
                                 buf.at[0], sem_i),
                pltpu.async_copy(dst2d.at[pl.ds(base + i * _K, _K), :],
                                 buf.at[1], sem_i)]

    idx_d = [None] * (_NOUT + 1)
    sc_d = [None] * _NOUT
    idx_d[0] = idx_start(0, ibuf[0])
    for i in range(_NOUT):
        rb = rows[i % 2]
        ib = ibuf[i % 3]
        if i >= 2:
            for d in sc_d[i - 2]:
                d.wait()
        for d in idx_d[i]:
            d.wait()
        if i + 1 < _NOUT:
            idx_d[i + 1] = idx_start(i + 1, ibuf[(i + 1) % 3])
        gs = [pltpu.async_copy(table.at[ib.at[0, j]], rb.at[j], sem_g)
              for j in range(_K)]
        for g in gs:
            g.wait()
        sc_d[i] = [pltpu.async_copy(rb.at[j], acc.at[ib.at[1, j]], sem_s,
                                    add=True)
                   for j in range(_K)]
    for i in (_NOUT - 2, _NOUT - 1):
        for d in sc_d[i]:
            d.wait()

    plsc.subcore_barrier()
    pltpu.sync_copy(acc.at[pl.ds(s * _RNODE, _RNODE), :],
                    out.at[pl.ds(c * _NP + s * _RNODE, _RNODE), :])


_BPK = 512
_GRIDK = _NPK // _BPK


def _tc0_body(xpk_ref, w1b_ref, out_ref):
    out_ref[...] = jnp.dot(xpk_ref[...], w1b_ref[...])


def _tc0(xpk, W1blk):
    return pl.pallas_call(
        _tc0_body,
        grid=(_GRIDK,),
        in_specs=[
            pl.BlockSpec((_BPK, 4 * _DIN), lambda i: (i, 0)),
            pl.BlockSpec((4 * _DIN, 128), lambda i: (0, 0)),
        ],
        out_specs=pl.BlockSpec((_BPK, 128), lambda i: (i, 0)),
        out_shape=jax.ShapeDtypeStruct((_NPK, 128), jnp.float32),
    )(xpk, W1blk)


def _tc1_body(dinvb_ref, xw_ref, out_ref):
    out_ref[...] = dinvb_ref[...] * xw_ref[...]


def _tc1(dinvb_p, xw):
    bs = pl.BlockSpec((_BPK, 128), lambda i: (i, 0))
    return pl.pallas_call(
        _tc1_body,
        grid=(_GRIDK,),
        in_specs=[bs, bs],
        out_specs=bs,
        out_shape=jax.ShapeDtypeStruct((_NPK, 128), jnp.float32),
    )(dinvb_p, xw)


def _tc2_body(a0_ref, a1_ref, xwp_ref, dinvb_ref, b1p_ref, w2b_ref, out_ref):
    d = dinvb_ref[...]
    h = d * (a0_ref[...] + a1_ref[...] + xwp_ref[...]) + b1p_ref[...]
    h = jnp.maximum(h, 0.0)
    out_ref[...] = d * jnp.dot(h, w2b_ref[...])


def _tc2(accs, xwp_p, dinvb_p, b1p, W2blk):
    bs = pl.BlockSpec((_BPK, 128), lambda i: (i, 0))
    return pl.pallas_call(
        _tc2_body,
        grid=(_GRIDK,),
        in_specs=[bs,
                  pl.BlockSpec((_BPK, 128), lambda i: (i + _GRIDK, 0)),
                  bs, bs,
                  pl.BlockSpec((1, 128), lambda i: (0, 0)),
                  pl.BlockSpec((128, 128), lambda i: (0, 0))],
        out_specs=bs,
        out_shape=jax.ShapeDtypeStruct((_NPK, 128), jnp.float32),
    )(accs, accs, xwp_p, dinvb_p, b1p, W2blk)


def _tc3_body(a0_ref, a1_ref, xwp_ref, dinvb_ref, b2p_ref, bi4_ref,
              wf1_ref, bf1_ref, wf2_ref, bf2_ref, out_ref):
    h2 = (dinvb_ref[...] * (a0_ref[...] + a1_ref[...] + xwp_ref[...])
          + b2p_ref[...])
    h2 = jnp.maximum(h2, 0.0)
    gidx = lax.broadcasted_iota(jnp.int32, (_NPK, _G), 1)
    ones = jnp.ones((_NPK, 1), jnp.float32)
    dn = (((0,), (0,)), ((), ()))
    hp = lax.Precision.HIGHEST
    ssum = jnp.zeros((_G, _DH), jnp.float32)
    cnt = jnp.zeros((_G, 1), jnp.float32)
    for q in range(4):
        oh = (bi4_ref[:, q:q + 1] == gidx).astype(jnp.float32)
        ssum = ssum + lax.dot_general(oh, h2[:, 32 * q:32 * q + 32], dn,
                                      precision=hp)
        cnt = cnt + lax.dot_general(oh, ones, dn, precision=hp)
    gemb = ssum / jnp.maximum(cnt, 1.0)
    z = jnp.dot(gemb, wf1_ref[...]) + bf1_ref[...]
    z = jnp.maximum(z, 0.0)
    out_ref[...] = jnp.dot(z, wf2_ref[...]) + bf2_ref[...]


def _tc3(accs, xwp_p, dinvb_p, b2p, bi4, Wf1, bf1, Wf2, bf2):
    half = pl.BlockSpec((_NPK, 128), lambda i: (0, 0))
    half1 = pl.BlockSpec((_NPK, 128), lambda i: (1, 0))
    return pl.pallas_call(
        _tc3_body,
        grid=(1,),
        in_specs=[half, half1] + [pl.BlockSpec(a.shape, lambda i: (0, 0))
                                  for a in (xwp_p, dinvb_p, b2p, bi4,
                                            Wf1, bf1, Wf2, bf2)],
        out_specs=pl.BlockSpec((_G, _DOUT), lambda i: (0, 0)),
        out_shape=jax.ShapeDtypeStruct((_G, _DOUT), jnp.float32),
    )(accs, accs, xwp_p, dinvb_p, b2p, bi4, Wf1, bf1, Wf2, bf2)


@jax.jit
def kernel(x, edge_index, batch_index, W1, b1, W2, b2, Wf1, bf1, Wf2, bf2):
    f32 = jnp.float32
    src2d = edge_index[0].reshape(_EROWS, _C)
    dst2d = edge_index[1].reshape(_EROWS, _C)

    pad = _NP - _N
    xpk = jnp.concatenate([x, jnp.zeros((pad, _DIN), f32)],
                          axis=0).reshape(_NPK, 4 * _DIN)
    bi4 = jnp.concatenate(
        [batch_index, jnp.full((pad,), _G, batch_index.dtype)]).reshape(
            _NPK, 4)
    eye4 = jnp.eye(4, dtype=f32)
    W1blk = jnp.kron(eye4, W1)
    W2blk = jnp.kron(eye4, W2)
    b1p = jnp.tile(b1, 4).reshape(1, 128)
    b2p = jnp.tile(b2, 4).reshape(1, 128)

    xw1 = _tc0(xpk, W1blk)
    dinvb_p = _sc_degree(dst2d)

    xwp1_p = _tc1(dinvb_p, xw1)
    acc1 = _sc_message(src2d, dst2d, xwp1_p.reshape(_NP, _DH))
    xwp2_p = _tc2(acc1.reshape(_NC * _NPK, 128), xwp1_p, dinvb_p, b1p, W2blk)
    acc2 = _sc_message(src2d, dst2d, xwp2_p.reshape(_NP, _DH))
    out = _tc3(acc2.reshape(_NC * _NPK, 128), xwp2_p, dinvb_p, b2p, bi4,
               Wf1, bf1.reshape(1, _DFC), Wf2, bf2.reshape(1, _DOUT))
    return out


# speedup vs baseline: 1.3190x; 1.0035x over previous
import functools

import jax
import jax.numpy as jnp
from jax import lax
from jax.experimental import pallas as pl
from jax.experimental.pallas import tpu as pltpu
from jax.experimental.pallas import tpu_sc as plsc

_N = 10000
_E = 320000
_DIN = 128
_DH = 32
_DOUT = 2
_G = 64
_DFC = 128

_NC = 2
_NS = 16
_NW = _NC * _NS

_C = 125
_K = 8
_EROWS = _E // _C
_RPT = _EROWS // _NW
_RPTF = _EROWS // _NS
_NOUT = _RPT // _K
_NOUTF = _RPTF // _K

_NP = 10240
_RNODE = _NP // _NS
_NPK = _NP // 4

_mesh = plsc.VectorSubcoreMesh(core_axis_name="c", subcore_axis_name="s")


@functools.partial(
    pl.kernel,
    out_type=jax.ShapeDtypeStruct((_NPK, 128), jnp.float32),
    mesh=_mesh,
    scratch_types=[
        [pltpu.VMEM((_K, _C), jnp.int32)] * 3,
        pltpu.VMEM((_RNODE,), jnp.float32),
        pltpu.VMEM((_RNODE,), jnp.float32),
        pltpu.VMEM((_RNODE // 4, 128), jnp.float32),
        pltpu.VMEM_SHARED((_NP,), jnp.float32),
        pltpu.SemaphoreType.DMA,
        pltpu.SemaphoreType.DMA,
    ],
    compiler_params=pltpu.CompilerParams(use_tc_tiling_on_sc=False,
                                         needs_layout_passes=False),
)
def _sc_degree(dst2d, dinvb, didx, ones_v, dbuf, dbv, acc, sem_i, sem_s):
    s = lax.axis_index("s")

    def zfill(j, carry):
        ones_v[pl.ds(j * 16, 16)] = jnp.zeros((16,), jnp.float32)
        return carry

    lax.fori_loop(0, _RNODE // 16, zfill, 0)
    pltpu.sync_copy(ones_v, acc.at[pl.ds(s * _RNODE, _RNODE)])
    for j in range(128 // 16):
        ones_v[pl.ds(j * 16, 16)] = jnp.ones((16,), jnp.float32)
    plsc.subcore_barrier()

    base = s * _RPTF
    ones_s = ones_v.at[pl.ds(0, _C)]

    idx_d = [None] * (_NOUTF + 1)
    sc_d = [None] * _NOUTF
    idx_d[0] = pltpu.async_copy(dst2d.at[pl.ds(base, _K), :], didx[0], sem_i)
    for i in range(_NOUTF):
        ib = didx[i % 3]
        if i >= 2:
            for d in sc_d[i - 2]:
                d.wait()
        idx_d[i].wait()
        if i + 1 < _NOUTF:
            idx_d[i + 1] = pltpu.async_copy(
                dst2d.at[pl.ds(base + (i + 1) * _K, _K), :],
                didx[(i + 1) % 3], sem_i)
        sc_d[i] = [pltpu.async_copy(ones_s, acc.at[ib.at[j]], sem_s,
                                    add=True)
                   for j in range(_K)]
    for i in (_NOUTF - 2, _NOUTF - 1):
        for d in sc_d[i]:
            d.wait()

    plsc.subcore_barrier()

    pltpu.sync_copy(acc.at[pl.ds(s * _RNODE, _RNODE)], dbuf)

    def newton(k, carry):
        v = dbuf[pl.ds(k * 16, 16)] + 1.0
        i0 = plsc.bitcast(v, jnp.int32)
        i0 = 0x5F3759DF - lax.shift_right_logical(i0, 1)
        y = plsc.bitcast(i0, jnp.float32)
        y = y * (1.5 - 0.5 * v * y * y)
        y = y * (1.5 - 0.5 * v * y * y)
        y = y * (1.5 - 0.5 * v * y * y)
        dbuf[pl.ds(k * 16, 16)] = y
        return carry

    lax.fori_loop(0, _RNODE // 16, newton, 0)

    def expand(k, carry):
        dv = dbuf[pl.ds(k * 16, 16)]
        for l in range(16):
            row = jnp.broadcast_to(dv[l], (16,))
            dbv[k * 4 + l // 4, pl.ds(32 * (l % 4), 16)] = row
            dbv[k * 4 + l // 4, pl.ds(32 * (l % 4) + 16, 16)] = row
        return carry

    lax.fori_loop(0, _RNODE // 16, expand, 0)
    pltpu.sync_copy(
        dbv, dinvb.at[pl.ds(s * (_RNODE // 4), _RNODE // 4), :])


@functools.partial(
    pl.kernel,
    out_type=jax.ShapeDtypeStruct((_NC * _NP, _DH), jnp.float32),
    mesh=_mesh,
    scratch_types=[
        [pltpu.VMEM((2, _K, _C), jnp.int32)] * 3,
        [pltpu.VMEM((_K, _C, _DH), jnp.float32)] * 2,
        pltpu.VMEM((80, _DH), jnp.float32),
        pltpu.VMEM_SHARED((_NP, _DH), jnp.float32),
        pltpu.SemaphoreType.DMA,
        pltpu.SemaphoreType.DMA,
        pltpu.SemaphoreType.DMA,
    ],
    compiler_params=pltpu.CompilerParams(use_tc_tiling_on_sc=False),
)
def _sc_message(src2d, dst2d, table, out, ibuf, rows, zbuf, acc,
                sem_i, sem_g, sem_s):
    c = lax.axis_index("c")
    s = lax.axis_index("s")
    wid = s * _NC + c

    def zfill(k, carry):
        zbuf[k, pl.ds(0, 16)] = jnp.zeros((16,), jnp.float32)
        zbuf[k, pl.ds(16, 16)] = jnp.zeros((16,), jnp.float32)
        return carry

    lax.fori_loop(0, 80, zfill, 0)
    for t in range(_RNODE // 80):
        pltpu.sync_copy(zbuf, acc.at[pl.ds(s * _RNODE + t * 80, 80), :])
    plsc.subcore_barrier()

    base = wid * _RPT

    def idx_start(i, buf):
        return [pltpu.async_copy(src2d.at[pl.ds(base + i * _K, _K), :],
                                 buf.at[0], sem_i),
                pltpu.async_copy(dst2d.at[pl.ds(base + i * _K, _K), :],
                                 buf.at[1], sem_i)]

    idx_d = [None] * (_NOUT + 1)
    sc_d = [None] * _NOUT
    idx_d[0] = idx_start(0, ibuf[0])
    for i in range(_NOUT):
        rb = rows[i % 2]
        ib = ibuf[i % 3]
        if i >= 2:
            for d in sc_d[i - 2]:
                d.wait()
        for d in idx_d[i]:
            d.wait()
        if i + 1 < _NOUT:
            idx_d[i + 1] = idx_start(i + 1, ibuf[(i + 1) % 3])
        gs = [pltpu.async_copy(table.at[ib.at[0, j]], rb.at[j], sem_g)
              for j in range(_K)]
        for g in gs:
            g.wait()
        sc_d[i] = [pltpu.async_copy(rb.at[j], acc.at[ib.at[1, j]], sem_s,
                                    add=True)
                   for j in range(_K)]
    for i in (_NOUT - 2, _NOUT - 1):
        for d in sc_d[i]:
            d.wait()

    plsc.subcore_barrier()
    pltpu.sync_copy(acc.at[pl.ds(s * _RNODE, _RNODE), :],
                    out.at[pl.ds(c * _NP + s * _RNODE, _RNODE), :])


_BPK = 512
_GRIDK = _NPK // _BPK


def _tc0_body(xpk_ref, w1b_ref, out_ref):
    out_ref[...] = jnp.dot(xpk_ref[...], w1b_ref[...])


def _tc0(xpk, W1blk):
    return pl.pallas_call(
        _tc0_body,
        grid=(_GRIDK,),
        in_specs=[
            pl.BlockSpec((_BPK, 4 * _DIN), lambda i: (i, 0)),
            pl.BlockSpec((4 * _DIN, 128), lambda i: (0, 0)),
        ],
        out_specs=pl.BlockSpec((_BPK, 128), lambda i: (i, 0)),
        out_shape=jax.ShapeDtypeStruct((_NPK, 128), jnp.float32),
    )(xpk, W1blk)


def _tc1_body(dinvb_ref, xw_ref, out_ref):
    out_ref[...] = dinvb_ref[...] * xw_ref[...]


def _tc1(dinvb_p, xw):
    bs = pl.BlockSpec((_BPK, 128), lambda i: (i, 0))
    return pl.pallas_call(
        _tc1_body,
        grid=(_GRIDK,),
        in_specs=[bs, bs],
        out_specs=bs,
        out_shape=jax.ShapeDtypeStruct((_NPK, 128), jnp.float32),
    )(dinvb_p, xw)


def _tc2_body(a0_ref, a1_ref, xwp_ref, dinvb_ref, b1p_ref, w2b_ref, out_ref):
    d = dinvb_ref[...]
    h = d * (a0_ref[...] + a1_ref[...] + xwp_ref[...]) + b1p_ref[...]
    h = jnp.maximum(h, 0.0)
    out_ref[...] = d * jnp.dot(h, w2b_ref[...])


def _tc2(accs, xwp_p, dinvb_p, b1p, W2blk):
    bs = pl.BlockSpec((_BPK, 128), lambda i: (i, 0))
    return pl.pallas_call(
        _tc2_body,
        grid=(_GRIDK,),
        in_specs=[bs,
                  pl.BlockSpec((_BPK, 128), lambda i: (i + _GRIDK, 0)),
                  bs, bs,
                  pl.BlockSpec((1, 128), lambda i: (0, 0)),
                  pl.BlockSpec((128, 128), lambda i: (0, 0))],
        out_specs=bs,
        out_shape=jax.ShapeDtypeStruct((_NPK, 128), jnp.float32),
    )(accs, accs, xwp_p, dinvb_p, b1p, W2blk)


def _tc3_body(a0_ref, a1_ref, xwp_ref, dinvb_ref, b2p_ref, bi4_ref,
              wf1_ref, bf1_ref, wf2_ref, bf2_ref, out_ref,
              ssum_ref, cnt_ref):
    i = pl.program_id(0)
    h2 = (dinvb_ref[...] * (a0_ref[...] + a1_ref[...] + xwp_ref[...])
          + b2p_ref[...])
    h2 = jnp.maximum(h2, 0.0)
    gidx = lax.broadcasted_iota(jnp.int32, (_BPK, _G), 1)
    ones = jnp.ones((_BPK, 1), jnp.float32)
    dn = (((0,), (0,)), ((), ()))
    hp = lax.Precision.HIGHEST
    ssum = jnp.zeros((_G, _DH), jnp.float32)
    cnt = jnp.zeros((_G, 1), jnp.float32)
    for q in range(4):
        oh = (bi4_ref[:, q:q + 1] == gidx).astype(jnp.float32)
        ssum = ssum + lax.dot_general(oh, h2[:, 32 * q:32 * q + 32], dn,
                                      precision=hp)
        cnt = cnt + lax.dot_general(oh, ones, dn, precision=hp)

    @pl.when(i == 0)
    def _():
        ssum_ref[...] = ssum
        cnt_ref[...] = cnt

    @pl.when(i > 0)
    def _():
        ssum_ref[...] += ssum
        cnt_ref[...] += cnt

    @pl.when(i == _GRIDK - 1)
    def _():
        gemb = ssum_ref[...] / jnp.maximum(cnt_ref[...], 1.0)
        z = jnp.dot(gemb, wf1_ref[...]) + bf1_ref[...]
        z = jnp.maximum(z, 0.0)
        out_ref[...] = jnp.dot(z, wf2_ref[...]) + bf2_ref[...]


def _tc3(accs, xwp_p, dinvb_p, b2p, bi4, Wf1, bf1, Wf2, bf2):
    bs = pl.BlockSpec((_BPK, 128), lambda i: (i, 0))
    return pl.pallas_call(
        _tc3_body,
        grid=(_GRIDK,),
        in_specs=[bs,
                  pl.BlockSpec((_BPK, 128), lambda i: (i + _GRIDK, 0)),
                  bs, bs,
                  pl.BlockSpec((1, 128), lambda i: (0, 0)),
                  pl.BlockSpec((_BPK, 4), lambda i: (i, 0)),
                  pl.BlockSpec(Wf1.shape, lambda i: (0, 0)),
                  pl.BlockSpec(bf1.shape, lambda i: (0, 0)),
                  pl.BlockSpec(Wf2.shape, lambda i: (0, 0)),
                  pl.BlockSpec(bf2.shape, lambda i: (0, 0))],
        out_specs=pl.BlockSpec((_G, _DOUT), lambda i: (0, 0)),
        out_shape=jax.ShapeDtypeStruct((_G, _DOUT), jnp.float32),
        scratch_shapes=[pltpu.VMEM((_G, _DH), jnp.float32),
                        pltpu.VMEM((_G, 1), jnp.float32)],
    )(accs, accs, xwp_p, dinvb_p, b2p, bi4, Wf1, bf1, Wf2, bf2)


@jax.jit
def kernel(x, edge_index, batch_index, W1, b1, W2, b2, Wf1, bf1, Wf2, bf2):
    f32 = jnp.float32
    src2d = edge_index[0].reshape(_EROWS, _C)
    dst2d = edge_index[1].reshape(_EROWS, _C)

    pad = _NP - _N
    xpk = jnp.concatenate([x, jnp.zeros((pad, _DIN), f32)],
                          axis=0).reshape(_NPK, 4 * _DIN)
    bi4 = jnp.concatenate(
        [batch_index, jnp.full((pad,), _G, batch_index.dtype)]).reshape(
            _NPK, 4)
    eye4 = jnp.eye(4, dtype=f32)
    W1blk = jnp.kron(eye4, W1)
    W2blk = jnp.kron(eye4, W2)
    b1p = jnp.tile(b1, 4).reshape(1, 128)
    b2p = jnp.tile(b2, 4).reshape(1, 128)

    xw1 = _tc0(xpk, W1blk)
    dinvb_p = _sc_degree(dst2d)

    xwp1_p = _tc1(dinvb_p, xw1)
    acc1 = _sc_message(src2d, dst2d, xwp1_p.reshape(_NP, _DH))
    xwp2_p = _tc2(acc1.reshape(_NC * _NPK, 128), xwp1_p, dinvb_p, b1p, W2blk)
    acc2 = _sc_message(src2d, dst2d, xwp2_p.reshape(_NP, _DH))
    out = _tc3(acc2.reshape(_NC * _NPK, 128), xwp2_p, dinvb_p, b2p, bi4,
               Wf1, bf1.reshape(1, _DFC), Wf2, bf2.reshape(1, _DOUT))
    return out


# speedup vs baseline: 1.3571x; 1.0289x over previous
import functools

import jax
import jax.numpy as jnp
from jax import lax
from jax.experimental import pallas as pl
from jax.experimental.pallas import tpu as pltpu
from jax.experimental.pallas import tpu_sc as plsc

_N = 10000
_E = 320000
_DIN = 128
_DH = 32
_DOUT = 2
_G = 64
_DFC = 128

_NC = 2
_NS = 16
_NW = _NC * _NS

_C = 125
_K = 8
_EROWS = _E // _C
_RPT = _EROWS // _NW
_RPTF = _EROWS // _NS
_NOUT = _RPT // _K
_NOUTF = _RPTF // _K

_NP = 10240
_RNODE = _NP // _NS
_NPK = _NP // 4

_mesh = plsc.VectorSubcoreMesh(core_axis_name="c", subcore_axis_name="s")


@functools.partial(
    pl.kernel,
    out_type=jax.ShapeDtypeStruct((_NC * _NP,), jnp.float32),
    mesh=_mesh,
    scratch_types=[
        [pltpu.VMEM((_K, _C), jnp.int32)] * 3,
        pltpu.VMEM((_RNODE,), jnp.float32),
        pltpu.VMEM_SHARED((_NP,), jnp.float32),
        pltpu.SemaphoreType.DMA,
        pltpu.SemaphoreType.DMA,
    ],
    compiler_params=pltpu.CompilerParams(use_tc_tiling_on_sc=False,
                                         needs_layout_passes=False),
)
def _sc_degree(dst2d, degp, didx, ones_v, acc, sem_i, sem_s):
    c = lax.axis_index("c")
    s = lax.axis_index("s")
    wid = s * _NC + c

    def zfill(j, carry):
        ones_v[pl.ds(j * 16, 16)] = jnp.zeros((16,), jnp.float32)
        return carry

    lax.fori_loop(0, _RNODE // 16, zfill, 0)
    pltpu.sync_copy(ones_v, acc.at[pl.ds(s * _RNODE, _RNODE)])
    for j in range(128 // 16):
        ones_v[pl.ds(j * 16, 16)] = jnp.ones((16,), jnp.float32)
    plsc.subcore_barrier()

    base = wid * _RPT
    ones_s = ones_v.at[pl.ds(0, _C)]

    idx_d = [None] * (_NOUT + 1)
    sc_d = [None] * _NOUT
    idx_d[0] = pltpu.async_copy(dst2d.at[pl.ds(base, _K), :], didx[0], sem_i)
    for i in range(_NOUT):
        ib = didx[i % 3]
        if i >= 2:
            for d in sc_d[i - 2]:
                d.wait()
        idx_d[i].wait()
        if i + 1 < _NOUT:
            idx_d[i + 1] = pltpu.async_copy(
                dst2d.at[pl.ds(base + (i + 1) * _K, _K), :],
                didx[(i + 1) % 3], sem_i)
        sc_d[i] = [pltpu.async_copy(ones_s, acc.at[ib.at[j]], sem_s,
                                    add=True)
                   for j in range(_K)]
    for i in (_NOUT - 2, _NOUT - 1):
        for d in sc_d[i]:
            d.wait()

    plsc.subcore_barrier()
    pltpu.sync_copy(acc.at[pl.ds(s * _RNODE, _RNODE)],
                    degp.at[pl.ds(c * _NP + s * _RNODE, _RNODE)])


@functools.partial(
    pl.kernel,
    out_type=jax.ShapeDtypeStruct((_NC * _NP, _DH), jnp.float32),
    mesh=_mesh,
    scratch_types=[
        [pltpu.VMEM((2, _K, _C), jnp.int32)] * 3,
        [pltpu.VMEM((_K, _C, _DH), jnp.float32)] * 2,
        pltpu.VMEM((80, _DH), jnp.float32),
        pltpu.VMEM_SHARED((_NP, _DH), jnp.float32),
        pltpu.SemaphoreType.DMA,
        pltpu.SemaphoreType.DMA,
        pltpu.SemaphoreType.DMA,
    ],
    compiler_params=pltpu.CompilerParams(use_tc_tiling_on_sc=False),
)
def _sc_message(src2d, dst2d, table, out, ibuf, rows, zbuf, acc,
                sem_i, sem_g, sem_s):
    c = lax.axis_index("c")
    s = lax.axis_index("s")
    wid = s * _NC + c

    def zfill(k, carry):
        zbuf[k, pl.ds(0, 16)] = jnp.zeros((16,), jnp.float32)
        zbuf[k, pl.ds(16, 16)] = jnp.zeros((16,), jnp.float32)
        return carry

    lax.fori_loop(0, 80, zfill, 0)
    for t in range(_RNODE // 80):
        pltpu.sync_copy(zbuf, acc.at[pl.ds(s * _RNODE + t * 80, 80), :])
    plsc.subcore_barrier()

    base = wid * _RPT

    def idx_start(i, buf):
        return [pltpu.async_copy(src2d.at[pl.ds(base + i * _K, _K), :],
                                 buf.at[0], sem_i),
                pltpu.async_copy(dst2d.at[pl.ds(base + i * _K, _K), :],
                                 buf.at[1], sem_i)]

    idx_d = [None] * (_NOUT + 1)
    sc_d = [None] * _NOUT
    idx_d[0] = idx_start(0, ibuf[0])
    for i in range(_NOUT):
        rb = rows[i % 2]
        ib = ibuf[i % 3]
        if i >= 2:
            for d in sc_d[i - 2]:
                d.wait()
        for d in idx_d[i]:
            d.wait()
        if i + 1 < _NOUT:
            idx_d[i + 1] = idx_start(i + 1, ibuf[(i + 1) % 3])
        gs = [pltpu.async_copy(table.at[ib.at[0, j]], rb.at[j], sem_g)
              for j in range(_K)]
        for g in gs:
            g.wait()
        sc_d[i] = [pltpu.async_copy(rb.at[j], acc.at[ib.at[1, j]], sem_s,
                                    add=True)
                   for j in range(_K)]
    for i in (_NOUT - 2, _NOUT - 1):
        for d in sc_d[i]:
            d.wait()

    plsc.subcore_barrier()
    pltpu.sync_copy(acc.at[pl.ds(s * _RNODE, _RNODE), :],
                    out.at[pl.ds(c * _NP + s * _RNODE, _RNODE), :])


_BPK = 512
_GRIDK = _NPK // _BPK


def _tc0_body(xpk_ref, w1b_ref, out_ref):
    out_ref[...] = jnp.dot(xpk_ref[...], w1b_ref[...])


def _tc0(xpk, W1blk):
    return pl.pallas_call(
        _tc0_body,
        grid=(_GRIDK,),
        in_specs=[
            pl.BlockSpec((_BPK, 4 * _DIN), lambda i: (i, 0)),
            pl.BlockSpec((4 * _DIN, 128), lambda i: (0, 0)),
        ],
        out_specs=pl.BlockSpec((_BPK, 128), lambda i: (i, 0)),
        out_shape=jax.ShapeDtypeStruct((_NPK, 128), jnp.float32),
    )(xpk, W1blk)


def _tc1_body(d0_ref, d1_ref, xw_ref, xwp_ref, dinvb_ref):
    deg = d0_ref[...] + d1_ref[...] + 1.0
    dinv = lax.rsqrt(deg)
    x_full = jnp.repeat(dinv, _BPK // 16, axis=0)
    b = lax.broadcasted_iota(jnp.int32, (_BPK, 128), 1)
    rr = lax.broadcasted_iota(jnp.int32, (_BPK, 128), 0)
    cbase = 4 * (rr % (_BPK // 16))
    cols = []
    for q in range(4):
        g = (b == cbase + q).astype(jnp.float32)
        t = jnp.sum(x_full * g, axis=1, keepdims=True)
        cols.append(jnp.broadcast_to(t, (_BPK, 32)))
    dinvb = jnp.concatenate(cols, axis=1)
    dinvb_ref[...] = dinvb
    xwp_ref[...] = dinvb * xw_ref[...]


def _tc1(degf, xw):
    bs = pl.BlockSpec((_BPK, 128), lambda i: (i, 0))
    return pl.pallas_call(
        _tc1_body,
        grid=(_GRIDK,),
        in_specs=[pl.BlockSpec((16, 128), lambda i: (i, 0)),
                  pl.BlockSpec((16, 128), lambda i: (i + _GRIDK, 0)),
                  bs],
        out_specs=[bs, bs],
        out_shape=[jax.ShapeDtypeStruct((_NPK, 128), jnp.float32),
                   jax.ShapeDtypeStruct((_NPK, 128), jnp.float32)],
    )(degf, degf, xw)


def _tc2_body(a0_ref, a1_ref, xwp_ref, dinvb_ref, b1p_ref, w2b_ref, out_ref):
    d = dinvb_ref[...]
    h = d * (a0_ref[...] + a1_ref[...] + xwp_ref[...]) + b1p_ref[...]
    h = jnp.maximum(h, 0.0)
    out_ref[...] = d * jnp.dot(h, w2b_ref[...])


def _tc2(accs, xwp_p, dinvb_p, b1p, W2blk):
    bs = pl.BlockSpec((_BPK, 128), lambda i: (i, 0))
    return pl.pallas_call(
        _tc2_body,
        grid=(_GRIDK,),
        in_specs=[bs,
                  pl.BlockSpec((_BPK, 128), lambda i: (i + _GRIDK, 0)),
                  bs, bs,
                  pl.BlockSpec((1, 128), lambda i: (0, 0)),
                  pl.BlockSpec((128, 128), lambda i: (0, 0))],
        out_specs=bs,
        out_shape=jax.ShapeDtypeStruct((_NPK, 128), jnp.float32),
    )(accs, accs, xwp_p, dinvb_p, b1p, W2blk)


def _tc3_body(a0_ref, a1_ref, xwp_ref, dinvb_ref, b2p_ref, bi4_ref,
              wf1_ref, bf1_ref, wf2_ref, bf2_ref, out_ref,
              ssum_ref, cnt_ref):
    i = pl.program_id(0)
    h2 = (dinvb_ref[...] * (a0_ref[...] + a1_ref[...] + xwp_ref[...])
          + b2p_ref[...])
    h2 = jnp.maximum(h2, 0.0)
    gidx = lax.broadcasted_iota(jnp.int32, (_BPK, _G), 1)
    ones = jnp.ones((_BPK, 1), jnp.float32)
    dn = (((0,), (0,)), ((), ()))
    hp = lax.Precision.HIGHEST
    ssum = jnp.zeros((_G, _DH), jnp.float32)
    cnt = jnp.zeros((_G, 1), jnp.float32)
    for q in range(4):
        oh = (bi4_ref[:, q:q + 1] == gidx).astype(jnp.float32)
        ssum = ssum + lax.dot_general(oh, h2[:, 32 * q:32 * q + 32], dn,
                                      precision=hp)
        cnt = cnt + lax.dot_general(oh, ones, dn, precision=hp)

    @pl.when(i == 0)
    def _():
        ssum_ref[...] = ssum
        cnt_ref[...] = cnt

    @pl.when(i > 0)
    def _():
        ssum_ref[...] += ssum
        cnt_ref[...] += cnt

    @pl.when(i == _GRIDK - 1)
    def _():
        gemb = ssum_ref[...] / jnp.maximum(cnt_ref[...], 1.0)
        z = jnp.dot(gemb, wf1_ref[...]) + bf1_ref[...]
        z = jnp.maximum(z, 0.0)
        out_ref[...] = jnp.dot(z, wf2_ref[...]) + bf2_ref[...]


def _tc3(accs, xwp_p, dinvb_p, b2p, bi4, Wf1, bf1, Wf2, bf2):
    bs = pl.BlockSpec((_BPK, 128), lambda i: (i, 0))
    return pl.pallas_call(
        _tc3_body,
        grid=(_GRIDK,),
        in_specs=[bs,
                  pl.BlockSpec((_BPK, 128), lambda i: (i + _GRIDK, 0)),
                  bs, bs,
                  pl.BlockSpec((1, 128), lambda i: (0, 0)),
                  pl.BlockSpec((_BPK, 4), lambda i: (i, 0)),
                  pl.BlockSpec(Wf1.shape, lambda i: (0, 0)),
                  pl.BlockSpec(bf1.shape, lambda i: (0, 0)),
                  pl.BlockSpec(Wf2.shape, lambda i: (0, 0)),
                  pl.BlockSpec(bf2.shape, lambda i: (0, 0))],
        out_specs=pl.BlockSpec((_G, _DOUT), lambda i: (0, 0)),
        out_shape=jax.ShapeDtypeStruct((_G, _DOUT), jnp.float32),
        scratch_shapes=[pltpu.VMEM((_G, _DH), jnp.float32),
                        pltpu.VMEM((_G, 1), jnp.float32)],
    )(accs, accs, xwp_p, dinvb_p, b2p, bi4, Wf1, bf1, Wf2, bf2)


@jax.jit
def kernel(x, edge_index, batch_index, W1, b1, W2, b2, Wf1, bf1, Wf2, bf2):
    f32 = jnp.float32
    src2d = edge_index[0].reshape(_EROWS, _C)
    dst2d = edge_index[1].reshape(_EROWS, _C)

    pad = _NP - _N
    xpk = jnp.concatenate([x, jnp.zeros((pad, _DIN), f32)],
                          axis=0).reshape(_NPK, 4 * _DIN)
    bi4 = jnp.concatenate(
        [batch_index, jnp.full((pad,), _G, batch_index.dtype)]).reshape(
            _NPK, 4)
    eye4 = jnp.eye(4, dtype=f32)
    W1blk = jnp.kron(eye4, W1)
    W2blk = jnp.kron(eye4, W2)
    b1p = jnp.tile(b1, 4).reshape(1, 128)
    b2p = jnp.tile(b2, 4).reshape(1, 128)

    xw1 = _tc0(xpk, W1blk)
    degp = _sc_degree(dst2d)

    xwp1_p, dinvb_p = _tc1(degp.reshape(_NC * _NP // 128, 128), xw1)
    acc1 = _sc_message(src2d, dst2d, xwp1_p.reshape(_NP, _DH))
    xwp2_p = _tc2(acc1.reshape(_NC * _NPK, 128), xwp1_p, dinvb_p, b1p, W2blk)
    acc2 = _sc_message(src2d, dst2d, xwp2_p.reshape(_NP, _DH))
    out = _tc3(acc2.reshape(_NC * _NPK, 128), xwp2_p, dinvb_p, b2p, bi4,
               Wf1, bf1.reshape(1, _DFC), Wf2, bf2.reshape(1, _DOUT))
    return out
